# baseline (speedup 1.0000x reference)
import functools

import jax
import jax.numpy as jnp
import numpy as np
from jax import lax
from jax.experimental import pallas as pl
from jax.experimental.pallas import tpu as pltpu
from jax.experimental.pallas import tpu_sc as plsc

_PERM = np.empty((128,), np.int32)
for _t in range(4):
    for _u in range(16):
        _PERM[32 * _t + 2 * _u] = 32 * _t + _u
        _PERM[32 * _t + 2 * _u + 1] = 32 * _t + 16 + _u

N_NODES = 10000
N_EDGES = 320000
DIM = 128
HALF = 64
EDGE_DIM = 16
N_ITERS = 3

NC = 2
NS = 16
EDGES_PER_TILE = N_EDGES // NS
CHUNK = 200
NCHUNKS = EDGES_PER_TILE // CHUNK
AGG_ROWS = 10240
ROWS_PER_TILE = AGG_ROWS // NS


def _sc_body(p_hbm, q_hbm, e_hbm, dst_hbm, src_hbm, out_hbm,
             di0, di1, di2, di3, si0, si1, si2, si3,
             gd0, gd1, gd2, gd3, gs0, gs1, gs2, gs3,
             pbuf0, pbuf1, qbuf0, qbuf1, ebuf0, ebuf1, mbuf0, mbuf1, aggs,
             isem0, isem1, isem2, isem3, sem0, sem1, ssem0, ssem1):
    cid = lax.axis_index("c")
    sid = lax.axis_index("s")
    di = (di0, di1, di2, di3)
    si = (si0, si1, si2, si3)
    gd = (gd0, gd1, gd2, gd3)
    gs = (gs0, gs1, gs2, gs3)
    isem = (isem0, isem1, isem2, isem3)
    pbuf = (pbuf0, pbuf1)
    qbuf = (qbuf0, qbuf1)
    ebuf = (ebuf0, ebuf1)
    mbuf = (mbuf0, mbuf1)
    sems = (sem0, sem1)
    ssem = (ssem0, ssem1)
    ebase = sid * EDGES_PER_TILE

    def _issue_idx(t, k):
        base = ebase + t * CHUNK
        pltpu.async_copy(dst_hbm.at[pl.ds(base, CHUNK)], di[k], isem[k])
        pltpu.async_copy(src_hbm.at[pl.ds(base, CHUNK)], si[k], isem[k])

    def _drain_idx(t, k):
        base = ebase + t * CHUNK
        pltpu.make_async_copy(dst_hbm.at[pl.ds(base, CHUNK)], di[k], isem[k]).wait()
        pltpu.make_async_copy(src_hbm.at[pl.ds(base, CHUNK)], si[k], isem[k]).wait()

    zero = jnp.zeros((16,), jnp.float32)

    def zrow(r, carry):
        for j in range(HALF // 16):
            mbuf0[r, pl.ds(j * 16, 16)] = zero
        return carry

    lax.fori_loop(0, CHUNK, zrow, 0)
    base_r = sid * ROWS_PER_TILE
    pltpu.sync_copy(mbuf0, aggs.at[pl.ds(base_r, CHUNK)])
    pltpu.sync_copy(mbuf0, aggs.at[pl.ds(base_r + CHUNK, CHUNK)])
    pltpu.sync_copy(mbuf0, aggs.at[pl.ds(base_r + 2 * CHUNK, CHUNK)])
    pltpu.sync_copy(mbuf0.at[pl.ds(0, ROWS_PER_TILE - 3 * CHUNK)],
                    aggs.at[pl.ds(base_r + 3 * CHUNK, ROWS_PER_TILE - 3 * CHUNK)])
    plsc.subcore_barrier()

    cof = cid * HALF

    def _xform(t, k):
        for g in range(CHUNK // 16 + 1):
            off = g * 16 if g <= CHUNK // 16 - 1 else CHUNK - 16
            sl = pl.ds(off, 16)
            gd[k][sl] = di[k][sl] * 2 + cid
            gs[k][sl] = si[k][sl] * 2 + cid

    def _issue(t, b, k):
        base = ebase + t * CHUNK
        _drain_idx(t, k)
        _xform(t, k)
        pltpu.async_copy(p_hbm.at[gd[k]], pbuf[b], sems[b])
        pltpu.async_copy(q_hbm.at[gs[k]], qbuf[b], sems[b])
        pltpu.async_copy(e_hbm.at[pl.ds(base, CHUNK), pl.ds(cof, HALF)],
                         ebuf[b], sems[b])

    def _drain(t, b, k):
        pltpu.make_async_copy(p_hbm.at[gd[k]], pbuf[b], sems[b]).wait()
        pltpu.make_async_copy(q_hbm.at[gs[k]], qbuf[b], sems[b]).wait()
        pltpu.make_async_copy(e_hbm.at[pl.ds(ebase, CHUNK), pl.ds(cof, HALF)],
                              ebuf[b], sems[b]).wait()

    _issue_idx(0, 0)
    _issue_idx(1, 1)
    _issue(0, 0, 0)

    def quad(qq, carry):
        for u in range(4):
            t = 4 * qq + u
            b = u % 2
            k = u

            @pl.when(t + 2 < NCHUNKS)
            def _():
                _issue_idx(t + 2, (u + 2) % 4)

            @pl.when(t + 1 < NCHUNKS)
            def _():
                @pl.when(t >= 1)
                def _():
                    pltpu.make_async_copy(
                        mbuf[1 - b], aggs.at[di[(u + 3) % 4]], ssem[1 - b]).wait()

                _issue(t + 1, 1 - b, (u + 1) % 4)

            _drain(t, b, k)

            @plsc.parallel_loop(0, CHUNK, 1, unroll=4)
            def _rows(r):
                for t2 in range(2):
                    slw = pl.ds(t2 * 16, 16)
                    pw = pbuf[b][r, slw]
                    qw = qbuf[b][r, slw]
                    pa = plsc.bitcast(lax.shift_left(pw, 16), jnp.float32)
                    qa = plsc.bitcast(lax.shift_left(qw, 16), jnp.float32)
                    pb2 = plsc.bitcast(jnp.bitwise_and(pw, jnp.int32(-65536)),
                                       jnp.float32)
                    qb2 = plsc.bitcast(jnp.bitwise_and(qw, jnp.int32(-65536)),
                                       jnp.float32)
                    mbuf[b][r, pl.ds(t2 * 32, 16)] = jnp.maximum(
                        pa + qa + ebuf[b][r, pl.ds(t2 * 32, 16)], 0.0)
                    mbuf[b][r, pl.ds(t2 * 32 + 16, 16)] = jnp.maximum(
                        pb2 + qb2 + ebuf[b][r, pl.ds(t2 * 32 + 16, 16)], 0.0)

            pltpu.async_copy(mbuf[b], aggs.at[di[k]], ssem[b], add=True)
        return carry

    lax.fori_loop(0, NCHUNKS // 4, quad, 0)
    pltpu.make_async_copy(mbuf[0], aggs.at[di[(NCHUNKS - 2) % 4]], ssem[0]).wait()
    pltpu.make_async_copy(mbuf[1], aggs.at[di[(NCHUNKS - 1) % 4]], ssem[1]).wait()
    plsc.subcore_barrier()
    pltpu.sync_copy(aggs.at[pl.ds(base_r, ROWS_PER_TILE)],
                    out_hbm.at[pl.ds(base_r, ROWS_PER_TILE), pl.ds(cof, HALF)])


_sc_edge = functools.partial(
    pl.kernel,
    out_type=jax.ShapeDtypeStruct((AGG_ROWS, DIM), jnp.float32),
    mesh=plsc.VectorSubcoreMesh(core_axis_name="c", subcore_axis_name="s",
                                num_cores=NC, num_subcores=NS),
    compiler_params=pltpu.CompilerParams(use_tc_tiling_on_sc=False,
                                         needs_layout_passes=False),
    scratch_types=(
        [pltpu.VMEM((CHUNK,), jnp.int32)] * 16
        + [pltpu.VMEM((CHUNK, HALF // 2), jnp.int32)] * 4
        + [pltpu.VMEM((CHUNK, HALF), jnp.float32)] * 4
        + [pltpu.VMEM_SHARED((AGG_ROWS, HALF), jnp.float32)]
        + [pltpu.SemaphoreType.DMA] * 8
    ),
)(_sc_body)


_ROWS_BLK = 2000
_NBLK = N_NODES // _ROWS_BLK
_EROWS_BLK = 10000
_NEBLK = N_EDGES // _EROWS_BLK


def _emb_body(x_ref, w_ref, b_ref, w12_ref, h_ref, p_ref, q_ref):
    h = jnp.maximum(
        jnp.dot(x_ref[...], w_ref[...], preferred_element_type=jnp.float32)
        + b_ref[...], 0.0)
    h_ref[...] = h
    p_ref[...] = jnp.dot(h, w12_ref[0:DIM, :],
                         preferred_element_type=jnp.float32).astype(jnp.bfloat16)
    q_ref[...] = jnp.dot(h, w12_ref[DIM:2 * DIM, :],
                         preferred_element_type=jnp.float32).astype(jnp.bfloat16)


def _emb(x, w, b, w12):
    return pl.pallas_call(
        _emb_body,
        grid=(_NBLK,),
        in_specs=[
            pl.BlockSpec((_ROWS_BLK, DIM), lambda i: (i, 0)),
            pl.BlockSpec((DIM, DIM), lambda i: (0, 0)),
            pl.BlockSpec((1, DIM), lambda i: (0, 0)),
            pl.BlockSpec((2 * DIM, DIM), lambda i: (0, 0)),
        ],
        out_specs=[
            pl.BlockSpec((_ROWS_BLK, DIM), lambda i: (i, 0)),
            pl.BlockSpec((_ROWS_BLK, DIM), lambda i: (i, 0)),
            pl.BlockSpec((_ROWS_BLK, DIM), lambda i: (i, 0)),
        ],
        out_shape=[
            jax.ShapeDtypeStruct((N_NODES, DIM), jnp.float32),
            jax.ShapeDtypeStruct((N_NODES, DIM), jnp.bfloat16),
            jax.ShapeDtypeStruct((N_NODES, DIM), jnp.bfloat16),
        ],
    )(x, w, b, w12)


def _epre_body(a_ref, w_ref, b_ref, o_ref):
    o_ref[...] = (jnp.dot(a_ref[...], w_ref[...],
                          preferred_element_type=jnp.float32) + b_ref[...])


def _epre(edge_attr, w3, b):
    return pl.pallas_call(
        _epre_body,
        grid=(_NEBLK,),
        in_specs=[
            pl.BlockSpec((_EROWS_BLK, EDGE_DIM), lambda i: (i, 0)),
            pl.BlockSpec((EDGE_DIM, DIM), lambda i: (0, 0)),
            pl.BlockSpec((1, DIM), lambda i: (0, 0)),
        ],
        out_specs=pl.BlockSpec((_EROWS_BLK, DIM), lambda i: (i, 0)),
        out_shape=jax.ShapeDtypeStruct((N_EDGES, DIM), jnp.float32),
    )(edge_attr, w3, b)


def _gru_body(h_ref, m_ref, agg_ref, wi_ref, wh_ref, bi_ref, bh_ref, w12_ref,
              ho_ref, mo_ref, p_ref, q_ref):
    m = m_ref[...] + agg_ref[...]
    h = h_ref[...]
    gi = jnp.dot(m, wi_ref[...], preferred_element_type=jnp.float32) + bi_ref[...]
    gh = jnp.dot(h, wh_ref[...], preferred_element_type=jnp.float32) + bh_ref[...]
    r = jax.nn.sigmoid(gi[:, 0:DIM] + gh[:, 0:DIM])
    z = jax.nn.sigmoid(gi[:, DIM:2 * DIM] + gh[:, DIM:2 * DIM])
    n = jnp.tanh(gi[:, 2 * DIM:3 * DIM] + r * gh[:, 2 * DIM:3 * DIM])
    hn = (1.0 - z) * n + z * h
    ho_ref[...] = hn
    mo_ref[...] = m
    p_ref[...] = jnp.dot(hn, w12_ref[0:DIM, :],
                         preferred_element_type=jnp.float32).astype(jnp.bfloat16)
    q_ref[...] = jnp.dot(hn, w12_ref[DIM:2 * DIM, :],
                         preferred_element_type=jnp.float32).astype(jnp.bfloat16)


def _gru(h, m, aggs, wi, wh, bi, bh, w12):
    return pl.pallas_call(
        _gru_body,
        grid=(_NBLK,),
        in_specs=[
            pl.BlockSpec((_ROWS_BLK, DIM), lambda i: (i, 0)),
            pl.BlockSpec((_ROWS_BLK, DIM), lambda i: (i, 0)),
            pl.BlockSpec((_ROWS_BLK, DIM), lambda i: (i, 0)),
            pl.BlockSpec((DIM, 3 * DIM), lambda i: (0, 0)),
            pl.BlockSpec((DIM, 3 * DIM), lambda i: (0, 0)),
            pl.BlockSpec((1, 3 * DIM), lambda i: (0, 0)),
            pl.BlockSpec((1, 3 * DIM), lambda i: (0, 0)),
            pl.BlockSpec((2 * DIM, DIM), lambda i: (0, 0)),
        ],
        out_specs=[
            pl.BlockSpec((_ROWS_BLK, DIM), lambda i: (i, 0)),
            pl.BlockSpec((_ROWS_BLK, DIM), lambda i: (i, 0)),
            pl.BlockSpec((_ROWS_BLK, DIM), lambda i: (i, 0)),
            pl.BlockSpec((_ROWS_BLK, DIM), lambda i: (i, 0)),
        ],
        out_shape=[
            jax.ShapeDtypeStruct((N_NODES, DIM), jnp.float32),
            jax.ShapeDtypeStruct((N_NODES, DIM), jnp.float32),
            jax.ShapeDtypeStruct((N_NODES, DIM), jnp.bfloat16),
            jax.ShapeDtypeStruct((N_NODES, DIM), jnp.bfloat16),
        ],
    )(h, m, aggs, wi, wh, bi, bh, w12)


def _readout_body(h_ref, w1_ref, b1_ref, w2_ref, b2_ref, o_ref):
    i = pl.program_id(0)
    part = jnp.maximum(
        jnp.dot(h_ref[...], w1_ref[...], preferred_element_type=jnp.float32)
        + b1_ref[...], 0.0)
    s = jnp.sum(part, axis=0, keepdims=True)
    blk = jnp.dot(s, w2_ref[...], preferred_element_type=jnp.float32)

    @pl.when(i == 0)
    def _():
        o_ref[...] = blk + float(N_NODES) * b2_ref[...]

    @pl.when(i != 0)
    def _():
        o_ref[...] = o_ref[...] + blk


def _readout(h, w1, b1, w2, b2):
    rh = w1.shape[1]
    return pl.pallas_call(
        _readout_body,
        grid=(_NBLK,),
        in_specs=[
            pl.BlockSpec((_ROWS_BLK, DIM), lambda i: (i, 0)),
            pl.BlockSpec((DIM, rh), lambda i: (0, 0)),
            pl.BlockSpec((1, rh), lambda i: (0, 0)),
            pl.BlockSpec((rh, DIM), lambda i: (0, 0)),
            pl.BlockSpec((1, DIM), lambda i: (0, 0)),
        ],
        out_specs=pl.BlockSpec((1, DIM), lambda i: (0, 0)),
        out_shape=jax.ShapeDtypeStruct((1, DIM), jnp.float32),
    )(h, w1, b1, w2, b2)


def kernel(x, edge_index, edge_attr, W_emb, b_emb, W_msg, b_msg,
           W_i, W_h, b_i, b_h, W_r1, b_r1, W_r2, b_r2):
    src = edge_index[0].astype(jnp.int32)
    dst = edge_index[1].astype(jnp.int32)
    w12 = W_msg[0:2 * DIM][:, _PERM]
    h, P, Q = _emb(x, W_emb, b_emb.reshape(1, -1), w12)
    E = _epre(edge_attr, W_msg[2 * DIM:2 * DIM + EDGE_DIM], b_msg.reshape(1, -1))
    m = jnp.zeros((N_NODES, DIM), jnp.float32)
    for _ in range(N_ITERS):
        Pi = lax.bitcast_convert_type(
            P.reshape(2 * N_NODES, HALF // 2, 2), jnp.int32)
        Qi = lax.bitcast_convert_type(
            Q.reshape(2 * N_NODES, HALF // 2, 2), jnp.int32)
        aggs = _sc_edge(Pi, Qi, E, dst, src)
        h, m, P, Q = _gru(h, m, aggs, W_i, W_h,
                          b_i.reshape(1, -1), b_h.reshape(1, -1), w12)
    out = _readout(h, W_r1, b_r1.reshape(1, -1), W_r2, b_r2.reshape(1, -1))
    return out.reshape(DIM)


# speedup vs baseline: 5.9021x; 5.9021x over previous
import functools

import jax
import jax.numpy as jnp
import numpy as np
from jax import lax
from jax.experimental import pallas as pl
from jax.experimental.pallas import tpu as pltpu
from jax.experimental.pallas import tpu_sc as plsc


N_NODES = 10000
N_EDGES = 320000
DIM = 128
HALF = 64
EDGE_DIM = 16
N_ITERS = 3

NC = 2
NS = 16
EDGES_PER_TILE = N_EDGES // NS
CHUNK = 200
NCHUNKS = EDGES_PER_TILE // CHUNK
AGG_ROWS = 10240
ROWS_PER_TILE = AGG_ROWS // NS


def _sc_body(pq_hbm, e_hbm, dst_hbm, src_hbm, out_hbm,
             di0, di1, di2, di3, si0, si1, si2, si3,
             gd0, gd1, gd2, gd3, gs0, gs1, gs2, gs3,
             pbuf0, pbuf1, qbuf0, qbuf1, ebuf0, ebuf1, mbuf0, mbuf1, aggs,
             isem0, isem1, isem2, isem3, sem0, sem1, ssem0, ssem1):
    cid = lax.axis_index("c")
    sid = lax.axis_index("s")
    di = (di0, di1, di2, di3)
    si = (si0, si1, si2, si3)
    gd = (gd0, gd1, gd2, gd3)
    gs = (gs0, gs1, gs2, gs3)
    isem = (isem0, isem1, isem2, isem3)
    pbuf = (pbuf0, pbuf1)
    qbuf = (qbuf0, qbuf1)
    ebuf = (ebuf0, ebuf1)
    mbuf = (mbuf0, mbuf1)
    sems = (sem0, sem1)
    ssem = (ssem0, ssem1)
    ebase = sid * EDGES_PER_TILE

    def _issue_idx(t, k):
        base = ebase + t * CHUNK
        pltpu.async_copy(dst_hbm.at[pl.ds(base, CHUNK)], di[k], isem[k])
        pltpu.async_copy(src_hbm.at[pl.ds(base, CHUNK)], si[k], isem[k])

    def _drain_idx(t, k):
        base = ebase + t * CHUNK
        pltpu.make_async_copy(dst_hbm.at[pl.ds(base, CHUNK)], di[k], isem[k]).wait()
        pltpu.make_async_copy(src_hbm.at[pl.ds(base, CHUNK)], si[k], isem[k]).wait()

    zero = jnp.zeros((16,), jnp.float32)

    def zrow(r, carry):
        for j in range(HALF // 16):
            mbuf0[r, pl.ds(j * 16, 16)] = zero
        return carry

    lax.fori_loop(0, CHUNK, zrow, 0)
    base_r = sid * ROWS_PER_TILE
    pltpu.sync_copy(mbuf0, aggs.at[pl.ds(base_r, CHUNK)])
    pltpu.sync_copy(mbuf0, aggs.at[pl.ds(base_r + CHUNK, CHUNK)])
    pltpu.sync_copy(mbuf0, aggs.at[pl.ds(base_r + 2 * CHUNK, CHUNK)])
    pltpu.sync_copy(mbuf0.at[pl.ds(0, ROWS_PER_TILE - 3 * CHUNK)],
                    aggs.at[pl.ds(base_r + 3 * CHUNK, ROWS_PER_TILE - 3 * CHUNK)])
    plsc.subcore_barrier()

    cof = cid * HALF

    def _xform(t, k):
        for g in range(CHUNK // 16 + 1):
            off = g * 16 if g <= CHUNK // 16 - 1 else CHUNK - 16
            sl = pl.ds(off, 16)
            gd[k][sl] = di[k][sl] * 4 + cid
            gs[k][sl] = si[k][sl] * 4 + (cid + 2)

    def _issue(t, b, k):
        base = ebase + t * CHUNK
        _drain_idx(t, k)
        _xform(t, k)
        pltpu.async_copy(pq_hbm.at[gd[k]], pbuf[b], sems[b])
        pltpu.async_copy(pq_hbm.at[gs[k]], qbuf[b], sems[b])
        pltpu.async_copy(e_hbm.at[pl.ds(base, CHUNK), pl.ds(cof, HALF)],
                         ebuf[b], sems[b])

    def _drain(t, b, k):
        pltpu.make_async_copy(pq_hbm.at[gd[k]], pbuf[b], sems[b]).wait()
        pltpu.make_async_copy(pq_hbm.at[gs[k]], qbuf[b], sems[b]).wait()
        pltpu.make_async_copy(e_hbm.at[pl.ds(ebase, CHUNK), pl.ds(cof, HALF)],
                              ebuf[b], sems[b]).wait()

    _issue_idx(0, 0)
    _issue_idx(1, 1)
    _issue(0, 0, 0)

    def quad(qq, carry):
        for u in range(4):
            t = 4 * qq + u
            b = u % 2
            k = u

            @pl.when(t + 2 < NCHUNKS)
            def _():
                _issue_idx(t + 2, (u + 2) % 4)

            @pl.when(t + 1 < NCHUNKS)
            def _():
                @pl.when(t >= 1)
                def _():
                    pltpu.make_async_copy(
                        mbuf[1 - b], aggs.at[di[(u + 3) % 4]], ssem[1 - b]).wait()

                _issue(t + 1, 1 - b, (u + 1) % 4)

            _drain(t, b, k)

            @plsc.parallel_loop(0, CHUNK, 1, unroll=4)
            def _rows(r):
                for g in range(2):
                    slw = pl.ds(g * 16, 16)
                    pw = pbuf[b][r, slw]
                    qw = qbuf[b][r, slw]
                    plo = plsc.bitcast(lax.shift_left(pw, 16), jnp.float32)
                    qlo = plsc.bitcast(lax.shift_left(qw, 16), jnp.float32)
                    phi = plsc.bitcast(jnp.bitwise_and(pw, jnp.int32(-65536)),
                                       jnp.float32)
                    qhi = plsc.bitcast(jnp.bitwise_and(qw, jnp.int32(-65536)),
                                       jnp.float32)
                    mbuf[b][r, pl.ds(g * 16, 16)] = jnp.maximum(
                        plo + qlo + ebuf[b][r, pl.ds(g * 16, 16)], 0.0)
                    mbuf[b][r, pl.ds(32 + g * 16, 16)] = jnp.maximum(
                        phi + qhi + ebuf[b][r, pl.ds(32 + g * 16, 16)], 0.0)

            pltpu.async_copy(mbuf[b], aggs.at[di[k]], ssem[b], add=True)
        return carry

    lax.fori_loop(0, NCHUNKS // 4, quad, 0)
    pltpu.make_async_copy(mbuf[0], aggs.at[di[(NCHUNKS - 2) % 4]], ssem[0]).wait()
    pltpu.make_async_copy(mbuf[1], aggs.at[di[(NCHUNKS - 1) % 4]], ssem[1]).wait()
    plsc.subcore_barrier()
    pltpu.sync_copy(aggs.at[pl.ds(base_r, ROWS_PER_TILE)],
                    out_hbm.at[pl.ds(base_r, ROWS_PER_TILE), pl.ds(cof, HALF)])


_sc_edge = functools.partial(
    pl.kernel,
    out_type=jax.ShapeDtypeStruct((AGG_ROWS, DIM), jnp.float32),
    mesh=plsc.VectorSubcoreMesh(core_axis_name="c", subcore_axis_name="s",
                                num_cores=NC, num_subcores=NS),
    compiler_params=pltpu.CompilerParams(use_tc_tiling_on_sc=False,
                                         needs_layout_passes=False),
    scratch_types=(
        [pltpu.VMEM((CHUNK,), jnp.int32)] * 16
        + [pltpu.VMEM((CHUNK, HALF // 2), jnp.int32)] * 4
        + [pltpu.VMEM((CHUNK, HALF), jnp.float32)] * 4
        + [pltpu.VMEM_SHARED((AGG_ROWS, HALF), jnp.float32)]
        + [pltpu.SemaphoreType.DMA] * 8
    ),
)(_sc_body)


_ROWS_BLK = 2000
_NBLK = N_NODES // _ROWS_BLK
_EROWS_BLK = 10000
_NEBLK = N_EDGES // _EROWS_BLK


def _pack_pq(p, q):
    def pk(x):
        xb = lax.bitcast_convert_type(
            x.astype(jnp.bfloat16).astype(jnp.float32), jnp.int32)
        lo = lax.shift_right_logical(xb, 16)
        hi = jnp.bitwise_and(xb, jnp.int32(-65536))
        return jnp.concatenate(
            [lo[:, 0:32] | hi[:, 32:64], lo[:, 64:96] | hi[:, 96:128]], axis=1)
    return jnp.concatenate([pk(p), pk(q)], axis=1)


def _emb_body(x_ref, w_ref, b_ref, w12_ref, h_ref, pq_ref):
    h = jnp.maximum(
        jnp.dot(x_ref[...], w_ref[...], preferred_element_type=jnp.float32)
        + b_ref[...], 0.0)
    h_ref[...] = h
    p = jnp.dot(h, w12_ref[0:DIM, :], preferred_element_type=jnp.float32)
    q = jnp.dot(h, w12_ref[DIM:2 * DIM, :], preferred_element_type=jnp.float32)
    pq_ref[...] = _pack_pq(p, q)


def _emb(x, w, b, w12):
    return pl.pallas_call(
        _emb_body,
        grid=(_NBLK,),
        in_specs=[
            pl.BlockSpec((_ROWS_BLK, DIM), lambda i: (i, 0)),
            pl.BlockSpec((DIM, DIM), lambda i: (0, 0)),
            pl.BlockSpec((1, DIM), lambda i: (0, 0)),
            pl.BlockSpec((2 * DIM, DIM), lambda i: (0, 0)),
        ],
        out_specs=[
            pl.BlockSpec((_ROWS_BLK, DIM), lambda i: (i, 0)),
            pl.BlockSpec((_ROWS_BLK, DIM), lambda i: (i, 0)),
        ],
        out_shape=[
            jax.ShapeDtypeStruct((N_NODES, DIM), jnp.float32),
            jax.ShapeDtypeStruct((N_NODES, DIM), jnp.int32),
        ],
    )(x, w, b, w12)


def _epre_body(a_ref, w_ref, b_ref, o_ref):
    o_ref[...] = (jnp.dot(a_ref[...], w_ref[...],
                          preferred_element_type=jnp.float32) + b_ref[...])


def _epre(edge_attr, w3, b):
    return pl.pallas_call(
        _epre_body,
        grid=(_NEBLK,),
        in_specs=[
            pl.BlockSpec((_EROWS_BLK, EDGE_DIM), lambda i: (i, 0)),
            pl.BlockSpec((EDGE_DIM, DIM), lambda i: (0, 0)),
            pl.BlockSpec((1, DIM), lambda i: (0, 0)),
        ],
        out_specs=pl.BlockSpec((_EROWS_BLK, DIM), lambda i: (i, 0)),
        out_shape=jax.ShapeDtypeStruct((N_EDGES, DIM), jnp.float32),
    )(edge_attr, w3, b)


def _gru_body(h_ref, m_ref, agg_ref, wi_ref, wh_ref, bi_ref, bh_ref, w12_ref,
              ho_ref, mo_ref, pq_ref):
    m = m_ref[...] + agg_ref[...]
    h = h_ref[...]
    gi = jnp.dot(m, wi_ref[...], preferred_element_type=jnp.float32) + bi_ref[...]
    gh = jnp.dot(h, wh_ref[...], preferred_element_type=jnp.float32) + bh_ref[...]
    r = jax.nn.sigmoid(gi[:, 0:DIM] + gh[:, 0:DIM])
    z = jax.nn.sigmoid(gi[:, DIM:2 * DIM] + gh[:, DIM:2 * DIM])
    n = jnp.tanh(gi[:, 2 * DIM:3 * DIM] + r * gh[:, 2 * DIM:3 * DIM])
    hn = (1.0 - z) * n + z * h
    ho_ref[...] = hn
    mo_ref[...] = m
    p = jnp.dot(hn, w12_ref[0:DIM, :], preferred_element_type=jnp.float32)
    q = jnp.dot(hn, w12_ref[DIM:2 * DIM, :], preferred_element_type=jnp.float32)
    pq_ref[...] = _pack_pq(p, q)


def _gru(h, m, aggs, wi, wh, bi, bh, w12):
    return pl.pallas_call(
        _gru_body,
        grid=(_NBLK,),
        in_specs=[
            pl.BlockSpec((_ROWS_BLK, DIM), lambda i: (i, 0)),
            pl.BlockSpec((_ROWS_BLK, DIM), lambda i: (i, 0)),
            pl.BlockSpec((_ROWS_BLK, DIM), lambda i: (i, 0)),
            pl.BlockSpec((DIM, 3 * DIM), lambda i: (0, 0)),
            pl.BlockSpec((DIM, 3 * DIM), lambda i: (0, 0)),
            pl.BlockSpec((1, 3 * DIM), lambda i: (0, 0)),
            pl.BlockSpec((1, 3 * DIM), lambda i: (0, 0)),
            pl.BlockSpec((2 * DIM, DIM), lambda i: (0, 0)),
        ],
        out_specs=[
            pl.BlockSpec((_ROWS_BLK, DIM), lambda i: (i, 0)),
            pl.BlockSpec((_ROWS_BLK, DIM), lambda i: (i, 0)),
            pl.BlockSpec((_ROWS_BLK, DIM), lambda i: (i, 0)),
        ],
        out_shape=[
            jax.ShapeDtypeStruct((N_NODES, DIM), jnp.float32),
            jax.ShapeDtypeStruct((N_NODES, DIM), jnp.float32),
            jax.ShapeDtypeStruct((N_NODES, DIM), jnp.int32),
        ],
    )(h, m, aggs, wi, wh, bi, bh, w12)


def _readout_body(h_ref, w1_ref, b1_ref, w2_ref, b2_ref, o_ref):
    i = pl.program_id(0)
    part = jnp.maximum(
        jnp.dot(h_ref[...], w1_ref[...], preferred_element_type=jnp.float32)
        + b1_ref[...], 0.0)
    s = jnp.sum(part, axis=0, keepdims=True)
    blk = jnp.dot(s, w2_ref[...], preferred_element_type=jnp.float32)

    @pl.when(i == 0)
    def _():
        o_ref[...] = blk + float(N_NODES) * b2_ref[...]

    @pl.when(i != 0)
    def _():
        o_ref[...] = o_ref[...] + blk


def _readout(h, w1, b1, w2, b2):
    rh = w1.shape[1]
    return pl.pallas_call(
        _readout_body,
        grid=(_NBLK,),
        in_specs=[
            pl.BlockSpec((_ROWS_BLK, DIM), lambda i: (i, 0)),
            pl.BlockSpec((DIM, rh), lambda i: (0, 0)),
            pl.BlockSpec((1, rh), lambda i: (0, 0)),
            pl.BlockSpec((rh, DIM), lambda i: (0, 0)),
            pl.BlockSpec((1, DIM), lambda i: (0, 0)),
        ],
        out_specs=pl.BlockSpec((1, DIM), lambda i: (0, 0)),
        out_shape=jax.ShapeDtypeStruct((1, DIM), jnp.float32),
    )(h, w1, b1, w2, b2)


def kernel(x, edge_index, edge_attr, W_emb, b_emb, W_msg, b_msg,
           W_i, W_h, b_i, b_h, W_r1, b_r1, W_r2, b_r2):
    src = edge_index[0].astype(jnp.int32)
    dst = edge_index[1].astype(jnp.int32)
    w12 = W_msg[0:2 * DIM]
    h, PQ = _emb(x, W_emb, b_emb.reshape(1, -1), w12)
    E = _epre(edge_attr, W_msg[2 * DIM:2 * DIM + EDGE_DIM], b_msg.reshape(1, -1))
    m = jnp.zeros((N_NODES, DIM), jnp.float32)
    for _ in range(N_ITERS):
        aggs = _sc_edge(PQ.reshape(4 * N_NODES, DIM // 4), E, dst, src)
        h, m, PQ = _gru(h, m, aggs, W_i, W_h,
                        b_i.reshape(1, -1), b_h.reshape(1, -1), w12)
    out = _readout(h, W_r1, b_r1.reshape(1, -1), W_r2, b_r2.reshape(1, -1))
    return out.reshape(DIM)


# speedup vs baseline: 6.2373x; 1.0568x over previous
import functools

import jax
import jax.numpy as jnp
import numpy as np
from jax import lax
from jax.experimental import pallas as pl
from jax.experimental.pallas import tpu as pltpu
from jax.experimental.pallas import tpu_sc as plsc


N_NODES = 10000
N_EDGES = 320000
DIM = 128
HALF = 64
EDGE_DIM = 16
N_ITERS = 3

NC = 2
NS = 16
EDGES_PER_TILE = N_EDGES // NS
CHUNK = 200
NCHUNKS = EDGES_PER_TILE // CHUNK
AGG_ROWS = 10240
ROWS_PER_TILE = AGG_ROWS // NS


def _sc_body(pq_hbm, e_hbm, dst_hbm, src_hbm, out_hbm,
             di0, di1, di2, di3, si0, si1, si2, si3,
             gd0, gd1, gd2, gd3, gs0, gs1, gs2, gs3,
             pbuf0, pbuf1, qbuf0, qbuf1, ebuf0, ebuf1, mbuf0, mbuf1, aggs,
             isem0, isem1, isem2, isem3, sem0, sem1, ssem0, ssem1):
    cid = lax.axis_index("c")
    sid = lax.axis_index("s")
    di = (di0, di1, di2, di3)
    si = (si0, si1, si2, si3)
    gd = (gd0, gd1, gd2, gd3)
    gs = (gs0, gs1, gs2, gs3)
    isem = (isem0, isem1, isem2, isem3)
    pbuf = (pbuf0, pbuf1)
    qbuf = (qbuf0, qbuf1)
    ebuf = (ebuf0, ebuf1)
    mbuf = (mbuf0, mbuf1)
    sems = (sem0, sem1)
    ssem = (ssem0, ssem1)
    ebase = sid * EDGES_PER_TILE

    def _issue_idx(t, k):
        base = ebase + t * CHUNK
        pltpu.async_copy(dst_hbm.at[pl.ds(base, CHUNK)], di[k], isem[k])
        pltpu.async_copy(src_hbm.at[pl.ds(base, CHUNK)], si[k], isem[k])

    def _drain_idx(t, k):
        base = ebase + t * CHUNK
        pltpu.make_async_copy(dst_hbm.at[pl.ds(base, CHUNK)], di[k], isem[k]).wait()
        pltpu.make_async_copy(src_hbm.at[pl.ds(base, CHUNK)], si[k], isem[k]).wait()

    zero = jnp.zeros((16,), jnp.float32)

    def zrow(r, carry):
        for j in range(HALF // 16):
            mbuf0[r, pl.ds(j * 16, 16)] = zero
        return carry

    lax.fori_loop(0, CHUNK, zrow, 0)
    base_r = sid * ROWS_PER_TILE
    pltpu.sync_copy(mbuf0, aggs.at[pl.ds(base_r, CHUNK)])
    pltpu.sync_copy(mbuf0, aggs.at[pl.ds(base_r + CHUNK, CHUNK)])
    pltpu.sync_copy(mbuf0, aggs.at[pl.ds(base_r + 2 * CHUNK, CHUNK)])
    pltpu.sync_copy(mbuf0.at[pl.ds(0, ROWS_PER_TILE - 3 * CHUNK)],
                    aggs.at[pl.ds(base_r + 3 * CHUNK, ROWS_PER_TILE - 3 * CHUNK)])
    plsc.subcore_barrier()

    ehalf = sid // (NS // 2)
    erow0 = ebase - ehalf * (N_EDGES // 2)
    ecol = cid * 32 + ehalf * 64

    def _xform(t, k):
        for g in range(CHUNK // 16 + 1):
            off = g * 16 if g <= CHUNK // 16 - 1 else CHUNK - 16
            sl = pl.ds(off, 16)
            gd[k][sl] = di[k][sl] * 4 + cid
            gs[k][sl] = si[k][sl] * 4 + (cid + 2)

    def _issue(t, b, k):
        _drain_idx(t, k)
        _xform(t, k)
        pltpu.async_copy(pq_hbm.at[gd[k]], pbuf[b], sems[b])
        pltpu.async_copy(pq_hbm.at[gs[k]], qbuf[b], sems[b])
        pltpu.async_copy(e_hbm.at[pl.ds(erow0 + t * CHUNK, CHUNK), pl.ds(ecol, 32)],
                         ebuf[b], sems[b])

    def _drain(t, b, k):
        pltpu.make_async_copy(pq_hbm.at[gd[k]], pbuf[b], sems[b]).wait()
        pltpu.make_async_copy(pq_hbm.at[gs[k]], qbuf[b], sems[b]).wait()
        pltpu.make_async_copy(e_hbm.at[pl.ds(erow0, CHUNK), pl.ds(ecol, 32)],
                              ebuf[b], sems[b]).wait()

    _issue_idx(0, 0)
    _issue_idx(1, 1)
    _issue(0, 0, 0)

    def quad(qq, carry):
        for u in range(4):
            t = 4 * qq + u
            b = u % 2
            k = u

            @pl.when(t + 2 < NCHUNKS)
            def _():
                _issue_idx(t + 2, (u + 2) % 4)

            @pl.when(t + 1 < NCHUNKS)
            def _():
                @pl.when(t >= 1)
                def _():
                    pltpu.make_async_copy(
                        mbuf[1 - b], aggs.at[di[(u + 3) % 4]], ssem[1 - b]).wait()

                _issue(t + 1, 1 - b, (u + 1) % 4)

            _drain(t, b, k)

            @plsc.parallel_loop(0, CHUNK, 1, unroll=4)
            def _rows(r):
                for g in range(2):
                    slw = pl.ds(g * 16, 16)
                    pw = pbuf[b][r, slw]
                    qw = qbuf[b][r, slw]
                    ew = ebuf[b][r, slw]
                    mask = jnp.int32(-65536)
                    plo = plsc.bitcast(lax.shift_left(pw, 16), jnp.float32)
                    qlo = plsc.bitcast(lax.shift_left(qw, 16), jnp.float32)
                    elo = plsc.bitcast(lax.shift_left(ew, 16), jnp.float32)
                    phi = plsc.bitcast(jnp.bitwise_and(pw, mask), jnp.float32)
                    qhi = plsc.bitcast(jnp.bitwise_and(qw, mask), jnp.float32)
                    ehi = plsc.bitcast(jnp.bitwise_and(ew, mask), jnp.float32)
                    mbuf[b][r, pl.ds(g * 16, 16)] = jnp.maximum(
                        plo + qlo + elo, 0.0)
                    mbuf[b][r, pl.ds(32 + g * 16, 16)] = jnp.maximum(
                        phi + qhi + ehi, 0.0)

            pltpu.async_copy(mbuf[b], aggs.at[di[k]], ssem[b], add=True)
        return carry

    lax.fori_loop(0, NCHUNKS // 4, quad, 0)
    pltpu.make_async_copy(mbuf[0], aggs.at[di[(NCHUNKS - 2) % 4]], ssem[0]).wait()
    pltpu.make_async_copy(mbuf[1], aggs.at[di[(NCHUNKS - 1) % 4]], ssem[1]).wait()
    plsc.subcore_barrier()
    pltpu.sync_copy(aggs.at[pl.ds(base_r, ROWS_PER_TILE)],
                    out_hbm.at[pl.ds(base_r, ROWS_PER_TILE), pl.ds(cid * HALF, HALF)])


_sc_edge = functools.partial(
    pl.kernel,
    out_type=jax.ShapeDtypeStruct((AGG_ROWS, DIM), jnp.float32),
    mesh=plsc.VectorSubcoreMesh(core_axis_name="c", subcore_axis_name="s",
                                num_cores=NC, num_subcores=NS),
    compiler_params=pltpu.CompilerParams(use_tc_tiling_on_sc=False,
                                         needs_layout_passes=False),
    scratch_types=(
        [pltpu.VMEM((CHUNK,), jnp.int32)] * 16
        + [pltpu.VMEM((CHUNK, HALF // 2), jnp.int32)] * 6
        + [pltpu.VMEM((CHUNK, HALF), jnp.float32)] * 2
        + [pltpu.VMEM_SHARED((AGG_ROWS, HALF), jnp.float32)]
        + [pltpu.SemaphoreType.DMA] * 8
    ),
)(_sc_body)


_ROWS_BLK = 2000
_NBLK = N_NODES // _ROWS_BLK
_EROWS_BLK = 10000
_NEBLK = N_EDGES // _EROWS_BLK


def _pack_pq(p, q):
    def pk(x):
        xb = lax.bitcast_convert_type(
            x.astype(jnp.bfloat16).astype(jnp.float32), jnp.int32)
        lo = lax.shift_right_logical(xb, 16)
        hi = jnp.bitwise_and(xb, jnp.int32(-65536))
        return jnp.concatenate(
            [lo[:, 0:32] | hi[:, 32:64], lo[:, 64:96] | hi[:, 96:128]], axis=1)
    return jnp.concatenate([pk(p), pk(q)], axis=1)


def _emb_body(x_ref, w_ref, b_ref, w12_ref, h_ref, pq_ref):
    h = jnp.maximum(
        jnp.dot(x_ref[...], w_ref[...], preferred_element_type=jnp.float32)
        + b_ref[...], 0.0)
    h_ref[...] = h
    p = jnp.dot(h, w12_ref[0:DIM, :], preferred_element_type=jnp.float32)
    q = jnp.dot(h, w12_ref[DIM:2 * DIM, :], preferred_element_type=jnp.float32)
    pq_ref[...] = _pack_pq(p, q)


def _emb(x, w, b, w12):
    return pl.pallas_call(
        _emb_body,
        grid=(_NBLK,),
        in_specs=[
            pl.BlockSpec((_ROWS_BLK, DIM), lambda i: (i, 0)),
            pl.BlockSpec((DIM, DIM), lambda i: (0, 0)),
            pl.BlockSpec((1, DIM), lambda i: (0, 0)),
            pl.BlockSpec((2 * DIM, DIM), lambda i: (0, 0)),
        ],
        out_specs=[
            pl.BlockSpec((_ROWS_BLK, DIM), lambda i: (i, 0)),
            pl.BlockSpec((_ROWS_BLK, DIM), lambda i: (i, 0)),
        ],
        out_shape=[
            jax.ShapeDtypeStruct((N_NODES, DIM), jnp.float32),
            jax.ShapeDtypeStruct((N_NODES, DIM), jnp.int32),
        ],
    )(x, w, b, w12)


_EH = N_EDGES // 2
_EB2 = _EROWS_BLK // 2


def _epre_body(a1_ref, a2_ref, w_ref, b_ref, o_ref):
    e1 = (jnp.dot(a1_ref[...], w_ref[...],
                  preferred_element_type=jnp.float32) + b_ref[...])
    e2 = (jnp.dot(a2_ref[...], w_ref[...],
                  preferred_element_type=jnp.float32) + b_ref[...])
    o_ref[...] = _pack_pq(e1, e2)


def _epre(edge_attr, w3, b):
    return pl.pallas_call(
        _epre_body,
        grid=(_NEBLK,),
        in_specs=[
            pl.BlockSpec((_EB2, EDGE_DIM), lambda i: (i, 0)),
            pl.BlockSpec((_EB2, EDGE_DIM), lambda i: (_EH // _EB2 + i, 0)),
            pl.BlockSpec((EDGE_DIM, DIM), lambda i: (0, 0)),
            pl.BlockSpec((1, DIM), lambda i: (0, 0)),
        ],
        out_specs=pl.BlockSpec((_EB2, DIM), lambda i: (i, 0)),
        out_shape=jax.ShapeDtypeStruct((_EH, DIM), jnp.int32),
    )(edge_attr, edge_attr, w3, b)


def _gru_body(h_ref, m_ref, agg_ref, wi_ref, wh_ref, bi_ref, bh_ref, w12_ref,
              ho_ref, mo_ref, pq_ref):
    m = m_ref[...] + agg_ref[...]
    h = h_ref[...]
    gi = jnp.dot(m, wi_ref[...], preferred_element_type=jnp.float32) + bi_ref[...]
    gh = jnp.dot(h, wh_ref[...], preferred_element_type=jnp.float32) + bh_ref[...]
    r = jax.nn.sigmoid(gi[:, 0:DIM] + gh[:, 0:DIM])
    z = jax.nn.sigmoid(gi[:, DIM:2 * DIM] + gh[:, DIM:2 * DIM])
    n = jnp.tanh(gi[:, 2 * DIM:3 * DIM] + r * gh[:, 2 * DIM:3 * DIM])
    hn = (1.0 - z) * n + z * h
    ho_ref[...] = hn
    mo_ref[...] = m
    p = jnp.dot(hn, w12_ref[0:DIM, :], preferred_element_type=jnp.float32)
    q = jnp.dot(hn, w12_ref[DIM:2 * DIM, :], preferred_element_type=jnp.float32)
    pq_ref[...] = _pack_pq(p, q)


def _gru(h, m, aggs, wi, wh, bi, bh, w12):
    return pl.pallas_call(
        _gru_body,
        grid=(_NBLK,),
        in_specs=[
            pl.BlockSpec((_ROWS_BLK, DIM), lambda i: (i, 0)),
            pl.BlockSpec((_ROWS_BLK, DIM), lambda i: (i, 0)),
            pl.BlockSpec((_ROWS_BLK, DIM), lambda i: (i, 0)),
            pl.BlockSpec((DIM, 3 * DIM), lambda i: (0, 0)),
            pl.BlockSpec((DIM, 3 * DIM), lambda i: (0, 0)),
            pl.BlockSpec((1, 3 * DIM), lambda i: (0, 0)),
            pl.BlockSpec((1, 3 * DIM), lambda i: (0, 0)),
            pl.BlockSpec((2 * DIM, DIM), lambda i: (0, 0)),
        ],
        out_specs=[
            pl.BlockSpec((_ROWS_BLK, DIM), lambda i: (i, 0)),
            pl.BlockSpec((_ROWS_BLK, DIM), lambda i: (i, 0)),
            pl.BlockSpec((_ROWS_BLK, DIM), lambda i: (i, 0)),
        ],
        out_shape=[
            jax.ShapeDtypeStruct((N_NODES, DIM), jnp.float32),
            jax.ShapeDtypeStruct((N_NODES, DIM), jnp.float32),
            jax.ShapeDtypeStruct((N_NODES, DIM), jnp.int32),
        ],
    )(h, m, aggs, wi, wh, bi, bh, w12)


def _readout_body(h_ref, w1_ref, b1_ref, w2_ref, b2_ref, o_ref):
    i = pl.program_id(0)
    part = jnp.maximum(
        jnp.dot(h_ref[...], w1_ref[...], preferred_element_type=jnp.float32)
        + b1_ref[...], 0.0)
    s = jnp.sum(part, axis=0, keepdims=True)
    blk = jnp.dot(s, w2_ref[...], preferred_element_type=jnp.float32)

    @pl.when(i == 0)
    def _():
        o_ref[...] = blk + float(N_NODES) * b2_ref[...]

    @pl.when(i != 0)
    def _():
        o_ref[...] = o_ref[...] + blk


def _readout(h, w1, b1, w2, b2):
    rh = w1.shape[1]
    return pl.pallas_call(
        _readout_body,
        grid=(_NBLK,),
        in_specs=[
            pl.BlockSpec((_ROWS_BLK, DIM), lambda i: (i, 0)),
            pl.BlockSpec((DIM, rh), lambda i: (0, 0)),
            pl.BlockSpec((1, rh), lambda i: (0, 0)),
            pl.BlockSpec((rh, DIM), lambda i: (0, 0)),
            pl.BlockSpec((1, DIM), lambda i: (0, 0)),
        ],
        out_specs=pl.BlockSpec((1, DIM), lambda i: (0, 0)),
        out_shape=jax.ShapeDtypeStruct((1, DIM), jnp.float32),
    )(h, w1, b1, w2, b2)


def kernel(x, edge_index, edge_attr, W_emb, b_emb, W_msg, b_msg,
           W_i, W_h, b_i, b_h, W_r1, b_r1, W_r2, b_r2):
    src = edge_index[0].astype(jnp.int32)
    dst = edge_index[1].astype(jnp.int32)
    w12 = W_msg[0:2 * DIM]
    h, PQ = _emb(x, W_emb, b_emb.reshape(1, -1), w12)
    E = _epre(edge_attr, W_msg[2 * DIM:2 * DIM + EDGE_DIM], b_msg.reshape(1, -1))
    m = jnp.zeros((N_NODES, DIM), jnp.float32)
    for _ in range(N_ITERS):
        aggs = _sc_edge(PQ.reshape(4 * N_NODES, DIM // 4), E, dst, src)
        h, m, PQ = _gru(h, m, aggs, W_i, W_h,
                        b_i.reshape(1, -1), b_h.reshape(1, -1), w12)
    out = _readout(h, W_r1, b_r1.reshape(1, -1), W_r2, b_r2.reshape(1, -1))
    return out.reshape(DIM)
